# baseline (speedup 1.0000x reference)
import functools

import jax
import jax.numpy as jnp
from jax import lax
from jax.experimental import pallas as pl
from jax.experimental.pallas import tpu as pltpu
from jax.experimental.pallas import tpu_sc as plsc

N = 10000
E = 320000
D = 128
H = D // 2
L = 16

NC = 2
NS = 16
EPT = E // NS
CHUNK = 80
NCHUNK = EPT // CHUNK
VGRP = CHUNK // L
NBUF_G = 4
NBUF_S = 2
NMAIN = (NCHUNK // NBUF_G) * NBUF_G
NTAIL = NCHUNK - NMAIN
ROWS_PER_TILE = 624
ROWS_REM = N - NS * ROWS_PER_TILE

_mesh = plsc.VectorSubcoreMesh(core_axis_name="c", subcore_axis_name="s")

_SPLAT_DNUMS = lax.GatherDimensionNumbers(
    offset_dims=(), collapsed_slice_dims=(0,), start_index_map=(0,))


def _splat(v, lane):
    idx = jnp.full((L, 1), lane, jnp.int32)
    return lax.gather(v, idx, _SPLAT_DNUMS, (1,),
                      mode=lax.GatherScatterMode.PROMISE_IN_BOUNDS)


@functools.partial(
    pl.kernel,
    out_type=jax.ShapeDtypeStruct((NC, N, H), jnp.float32),
    mesh=_mesh,
    compiler_params=pltpu.CompilerParams(use_tc_tiling_on_sc=False),
    scratch_types=[
        pltpu.VMEM((NCHUNK, CHUNK), jnp.int32),
        pltpu.VMEM((NCHUNK, CHUNK), jnp.int32),
        pltpu.VMEM((EPT,), jnp.float32),
        [pltpu.VMEM((CHUNK, H), jnp.float32)] * NBUF_G,
        [pltpu.VMEM((CHUNK, H), jnp.float32)] * NBUF_S,
        pltpu.VMEM_SHARED((N, H), jnp.float32),
        [pltpu.SemaphoreType.DMA] * NBUF_G,
        [pltpu.SemaphoreType.DMA] * NBUF_S,
    ],
)
def _spmm_sc(hlo_hbm, hhi_hbm, src_hbm, dst_hbm, vals_hbm, zeros_hbm, out_hbm,
             src_v, dst_v, vals_v, grow, srow, acc, gsem, ssem):
    cid = lax.axis_index("c")
    sid = lax.axis_index("s")

    pltpu.sync_copy(src_hbm.at[sid], src_v)
    pltpu.sync_copy(dst_hbm.at[sid], dst_v)
    pltpu.sync_copy(vals_hbm.at[sid], vals_v)

    pltpu.sync_copy(zeros_hbm.at[pl.ds(sid * ROWS_PER_TILE, ROWS_PER_TILE)],
                    acc.at[pl.ds(sid * ROWS_PER_TILE, ROWS_PER_TILE)])

    @pl.when(sid == 0)
    def _():
        pltpu.sync_copy(zeros_hbm.at[pl.ds(NS * ROWS_PER_TILE, ROWS_REM)],
                        acc.at[pl.ds(NS * ROWS_PER_TILE, ROWS_REM)])

    plsc.subcore_barrier()

    def run(h_hbm):
        def chunk_body(jj, bg, bs, prefetch):
            pltpu.make_async_copy(h_hbm.at[src_v.at[jj]], grow[bg],
                                  gsem[bg]).wait()

            def _wait_scatter():
                pltpu.make_async_copy(srow[bs], acc.at[dst_v.at[jj]],
                                      ssem[bs]).wait()

            if isinstance(jj, int):
                if jj >= NBUF_S:
                    _wait_scatter()
            else:
                pl.when(jj >= NBUF_S)(_wait_scatter)

            base = jj * CHUNK
            for g in range(VGRP):
                v = vals_v[pl.ds(base + g * L, L)]
                for r in range(L):
                    row = g * L + r
                    s = _splat(v, r)
                    for q in range(H // L):
                        sl = pl.ds(q * L, L)
                        srow[bs][row, sl] = grow[bg][row, sl] * s

            pltpu.async_copy(srow[bs], acc.at[dst_v.at[jj]], ssem[bs],
                             add=True)

            if prefetch:
                @pl.when(jj + NBUF_G < NCHUNK)
                def _():
                    pltpu.async_copy(h_hbm.at[src_v.at[jj + NBUF_G]],
                                     grow[bg], gsem[bg])

        for b in range(NBUF_G):
            pltpu.async_copy(h_hbm.at[src_v.at[b]], grow[b], gsem[b])

        @pl.loop(0, NMAIN, step=NBUF_G)
        def _(j):
            for b in range(NBUF_G):
                chunk_body(j + b, b, b % NBUF_S, True)

        for t in range(NTAIL):
            jj = NMAIN + t
            chunk_body(jj, jj % NBUF_G, jj % NBUF_S, False)

        for b in range(NBUF_S):
            jj = NCHUNK - NBUF_S + b
            pltpu.make_async_copy(srow[jj % NBUF_S], acc.at[dst_v.at[jj]],
                                  ssem[jj % NBUF_S]).wait()

    @pl.when(cid == 0)
    def _():
        run(hlo_hbm)

    @pl.when(cid == 1)
    def _():
        run(hhi_hbm)

    plsc.subcore_barrier()
    pltpu.sync_copy(acc.at[pl.ds(sid * ROWS_PER_TILE, ROWS_PER_TILE)],
                    out_hbm.at[cid, pl.ds(sid * ROWS_PER_TILE, ROWS_PER_TILE)])

    @pl.when(sid == 0)
    def _():
        pltpu.sync_copy(acc.at[pl.ds(NS * ROWS_PER_TILE, ROWS_REM)],
                        out_hbm.at[cid, pl.ds(NS * ROWS_PER_TILE, ROWS_REM)])


def _mm_body(x_ref, w_ref, lo_ref, hi_ref):
    h = jnp.dot(x_ref[...], w_ref[...], preferred_element_type=jnp.float32)
    lo_ref[...] = h[:, :H]
    hi_ref[...] = h[:, H:]


def _mm_combine_body(p_ref, w_ref, lo_ref, hi_ref):
    x = jnp.concatenate([jnp.maximum(p_ref[0], 0.0),
                         jnp.maximum(p_ref[1], 0.0)], axis=1)
    h = jnp.dot(x, w_ref[...], preferred_element_type=jnp.float32)
    lo_ref[...] = h[:, :H]
    hi_ref[...] = h[:, H:]


def _combine_body(q_ref, o_ref):
    o_ref[:, :H] = jnp.maximum(q_ref[0], 0.0)
    o_ref[:, H:] = jnp.maximum(q_ref[1], 0.0)


_half_shapes = [jax.ShapeDtypeStruct((N, H), jnp.float32)] * 2
_mm = pl.pallas_call(_mm_body, out_shape=_half_shapes)
_mm_combine = pl.pallas_call(_mm_combine_body, out_shape=_half_shapes)
_combine = pl.pallas_call(
    _combine_body, out_shape=jax.ShapeDtypeStruct((N, D), jnp.float32))


def kernel(node_feats, edge_index, edge_vals, nodes_mask, W0, W1):
    src = edge_index[0].reshape(NS, NCHUNK, CHUNK)
    dst = edge_index[1].reshape(NS, NCHUNK, CHUNK)
    vals = edge_vals.reshape(NS, EPT)
    zeros = jnp.zeros((N, H), jnp.float32)

    h0_lo, h0_hi = _mm(node_feats, W0)
    p = _spmm_sc(h0_lo, h0_hi, src, dst, vals, zeros)
    h1_lo, h1_hi = _mm_combine(p, W1)
    q = _spmm_sc(h1_lo, h1_hi, src, dst, vals, zeros)
    return _combine(q)


# speedup vs baseline: 1.0412x; 1.0412x over previous
import functools

import jax
import jax.numpy as jnp
from jax import lax
from jax.experimental import pallas as pl
from jax.experimental.pallas import tpu as pltpu
from jax.experimental.pallas import tpu_sc as plsc

N = 10000
E = 320000
D = 128
H = D // 2
L = 16

NC = 2
NS = 16
EPT = E // NS
CHUNK = 80
NCHUNK = EPT // CHUNK
VGRP = CHUNK // L
NHALF = NCHUNK // 2
EHALF = EPT // 2
NBUF_G = 2
NBUF_S = 2
NMAIN = (NHALF // NBUF_G) * NBUF_G
NTAIL = NHALF - NMAIN
ROWS_PER_TILE = 624
ROWS_REM = N - NS * ROWS_PER_TILE

_mesh = plsc.VectorSubcoreMesh(core_axis_name="c", subcore_axis_name="s")

_SPLAT_DNUMS = lax.GatherDimensionNumbers(
    offset_dims=(), collapsed_slice_dims=(0,), start_index_map=(0,))


def _splat(v, lane):
    idx = jnp.full((L, 1), lane, jnp.int32)
    return lax.gather(v, idx, _SPLAT_DNUMS, (1,),
                      mode=lax.GatherScatterMode.PROMISE_IN_BOUNDS)


@functools.partial(
    pl.kernel,
    out_type=jax.ShapeDtypeStruct((NC, N, H), jnp.float32),
    mesh=_mesh,
    compiler_params=pltpu.CompilerParams(use_tc_tiling_on_sc=False),
    scratch_types=[
        pltpu.VMEM((NHALF, CHUNK), jnp.int32),
        pltpu.VMEM((NHALF, CHUNK), jnp.int32),
        pltpu.VMEM((EHALF,), jnp.float32),
        [pltpu.VMEM((CHUNK, H), jnp.float32)] * NBUF_G,
        [pltpu.VMEM((CHUNK, H), jnp.float32)] * NBUF_S,
        pltpu.VMEM_SHARED((N, H), jnp.float32),
        pltpu.VMEM_SHARED((N, H), jnp.float32),
        [pltpu.SemaphoreType.DMA] * NBUF_G,
        [pltpu.SemaphoreType.DMA] * NBUF_S,
    ],
)
def _spmm_sc(hlo_hbm, hhi_hbm, src_hbm, dst_hbm, vals_hbm, zeros_hbm, out_hbm,
             src_v, dst_v, vals_v, grow, srow, table, acc, gsem, ssem):
    cid = lax.axis_index("c")
    sid = lax.axis_index("s")

    rows = pl.ds(sid * ROWS_PER_TILE, ROWS_PER_TILE)
    rem = pl.ds(NS * ROWS_PER_TILE, ROWS_REM)

    pltpu.sync_copy(zeros_hbm.at[rows], acc.at[rows])

    @pl.when(cid == 0)
    def _():
        pltpu.sync_copy(hlo_hbm.at[rows], table.at[rows])

    @pl.when(cid == 1)
    def _():
        pltpu.sync_copy(hhi_hbm.at[rows], table.at[rows])

    @pl.when(sid == 0)
    def _():
        pltpu.sync_copy(zeros_hbm.at[rem], acc.at[rem])

        @pl.when(cid == 0)
        def _():
            pltpu.sync_copy(hlo_hbm.at[rem], table.at[rem])

        @pl.when(cid == 1)
        def _():
            pltpu.sync_copy(hhi_hbm.at[rem], table.at[rem])

    plsc.subcore_barrier()

    def chunk_body(jj, bg, bs, prefetch):
        pltpu.make_async_copy(table.at[src_v.at[jj]], grow[bg],
                              gsem[bg]).wait()

        def _wait_scatter():
            pltpu.make_async_copy(srow[bs], acc.at[dst_v.at[jj]],
                                  ssem[bs]).wait()

        if isinstance(jj, int):
            if jj >= NBUF_S:
                _wait_scatter()
        else:
            pl.when(jj >= NBUF_S)(_wait_scatter)

        base = jj * CHUNK
        for g in range(VGRP):
            v = vals_v[pl.ds(base + g * L, L)]
            for r in range(L):
                row = g * L + r
                s = _splat(v, r)
                for q in range(H // L):
                    sl = pl.ds(q * L, L)
                    srow[bs][row, sl] = grow[bg][row, sl] * s

        pltpu.async_copy(srow[bs], acc.at[dst_v.at[jj]], ssem[bs],
                         add=True)

        if prefetch:
            @pl.when(jj + NBUF_G < NHALF)
            def _():
                pltpu.async_copy(table.at[src_v.at[jj + NBUF_G]],
                                 grow[bg], gsem[bg])

    for half in range(2):
        pltpu.sync_copy(src_hbm.at[sid, pl.ds(half * NHALF, NHALF)], src_v)
        pltpu.sync_copy(dst_hbm.at[sid, pl.ds(half * NHALF, NHALF)], dst_v)
        pltpu.sync_copy(vals_hbm.at[sid, pl.ds(half * EHALF, EHALF)], vals_v)

        for b in range(NBUF_G):
            pltpu.async_copy(table.at[src_v.at[b]], grow[b], gsem[b])

        @pl.loop(0, NMAIN, step=NBUF_G)
        def _(j):
            for b in range(NBUF_G):
                chunk_body(j + b, b, b % NBUF_S, True)

        for t in range(NTAIL):
            jj = NMAIN + t
            chunk_body(jj, jj % NBUF_G, jj % NBUF_S, False)

        for b in range(NBUF_S):
            jj = NHALF - NBUF_S + b
            pltpu.make_async_copy(srow[jj % NBUF_S], acc.at[dst_v.at[jj]],
                                  ssem[jj % NBUF_S]).wait()

    plsc.subcore_barrier()
    pltpu.sync_copy(acc.at[rows], out_hbm.at[cid, rows])

    @pl.when(sid == 0)
    def _():
        pltpu.sync_copy(acc.at[rem], out_hbm.at[cid, rem])


def _mm_body(x_ref, w_ref, lo_ref, hi_ref):
    h = jnp.dot(x_ref[...], w_ref[...], preferred_element_type=jnp.float32)
    lo_ref[...] = h[:, :H]
    hi_ref[...] = h[:, H:]


def _mm_combine_body(p_ref, w_ref, lo_ref, hi_ref):
    x = jnp.concatenate([jnp.maximum(p_ref[0], 0.0),
                         jnp.maximum(p_ref[1], 0.0)], axis=1)
    h = jnp.dot(x, w_ref[...], preferred_element_type=jnp.float32)
    lo_ref[...] = h[:, :H]
    hi_ref[...] = h[:, H:]


def _combine_body(q_ref, o_ref):
    o_ref[:, :H] = jnp.maximum(q_ref[0], 0.0)
    o_ref[:, H:] = jnp.maximum(q_ref[1], 0.0)


_half_shapes = [jax.ShapeDtypeStruct((N, H), jnp.float32)] * 2
_mm = pl.pallas_call(_mm_body, out_shape=_half_shapes)
_mm_combine = pl.pallas_call(_mm_combine_body, out_shape=_half_shapes)
_combine = pl.pallas_call(
    _combine_body, out_shape=jax.ShapeDtypeStruct((N, D), jnp.float32))


def kernel(node_feats, edge_index, edge_vals, nodes_mask, W0, W1):
    src = edge_index[0].reshape(NS, NCHUNK, CHUNK)
    dst = edge_index[1].reshape(NS, NCHUNK, CHUNK)
    vals = edge_vals.reshape(NS, EPT)
    zeros = jnp.zeros((N, H), jnp.float32)

    h0_lo, h0_hi = _mm(node_feats, W0)
    p = _spmm_sc(h0_lo, h0_hi, src, dst, vals, zeros)
    h1_lo, h1_hi = _mm_combine(p, W1)
    q = _spmm_sc(h1_lo, h1_hi, src, dst, vals, zeros)
    return _combine(q)


# speedup vs baseline: 1.1014x; 1.0579x over previous
import functools

import jax
import jax.numpy as jnp
from jax import lax
from jax.experimental import pallas as pl
from jax.experimental.pallas import tpu as pltpu
from jax.experimental.pallas import tpu_sc as plsc

N = 10000
E = 320000
D = 128
L = 16

NC = 2
NS = 16
EPT = E // (NC * NS)
CHUNK = 80
NCHUNK = EPT // CHUNK
VGRP = CHUNK // L
NSTG = 5
SCH = NCHUNK // NSTG
NBUF_G = 2
NBUF_S = 2
NMAIN = (SCH // NBUF_G) * NBUF_G
NTAIL = SCH - NMAIN
ROWS_PER_TILE = 624
ROWS_REM = N - NS * ROWS_PER_TILE

_mesh = plsc.VectorSubcoreMesh(core_axis_name="c", subcore_axis_name="s")

_SPLAT_DNUMS = lax.GatherDimensionNumbers(
    offset_dims=(), collapsed_slice_dims=(0,), start_index_map=(0,))


def _splat(v, lane):
    idx = jnp.full((L, 1), lane, jnp.int32)
    return lax.gather(v, idx, _SPLAT_DNUMS, (1,),
                      mode=lax.GatherScatterMode.PROMISE_IN_BOUNDS)


@functools.partial(
    pl.kernel,
    out_type=jax.ShapeDtypeStruct((NC, N, D), jnp.float32),
    mesh=_mesh,
    compiler_params=pltpu.CompilerParams(use_tc_tiling_on_sc=False),
    scratch_types=[
        pltpu.VMEM((SCH, CHUNK), jnp.int32),
        pltpu.VMEM((SCH, CHUNK), jnp.int32),
        pltpu.VMEM((SCH * CHUNK,), jnp.float32),
        [pltpu.VMEM((CHUNK, D), jnp.float32)] * NBUF_G,
        [pltpu.VMEM((CHUNK, D), jnp.float32)] * NBUF_S,
        pltpu.VMEM_SHARED((N, D), jnp.float32),
        [pltpu.SemaphoreType.DMA] * NBUF_G,
        [pltpu.SemaphoreType.DMA] * NBUF_S,
    ],
)
def _spmm_sc(h_hbm, src_hbm, dst_hbm, vals_hbm, zeros_hbm, out_hbm,
             src_v, dst_v, vals_v, grow, srow, acc, gsem, ssem):
    cid = lax.axis_index("c")
    sid = lax.axis_index("s")

    rows = pl.ds(sid * ROWS_PER_TILE, ROWS_PER_TILE)
    rem = pl.ds(NS * ROWS_PER_TILE, ROWS_REM)

    pltpu.sync_copy(zeros_hbm.at[rows], acc.at[rows])

    @pl.when(sid == 0)
    def _():
        pltpu.sync_copy(zeros_hbm.at[rem], acc.at[rem])

    plsc.subcore_barrier()

    def chunk_body(jj, bg, bs, prefetch):
        pltpu.make_async_copy(h_hbm.at[src_v.at[jj]], grow[bg],
                              gsem[bg]).wait()

        def _wait_scatter():
            pltpu.make_async_copy(srow[bs], acc.at[dst_v.at[jj]],
                                  ssem[bs]).wait()

        if isinstance(jj, int):
            if jj >= NBUF_S:
                _wait_scatter()
        else:
            pl.when(jj >= NBUF_S)(_wait_scatter)

        base = jj * CHUNK
        for g in range(VGRP):
            v = vals_v[pl.ds(base + g * L, L)]
            for r in range(L):
                row = g * L + r
                s = _splat(v, r)
                for q in range(D // L):
                    sl = pl.ds(q * L, L)
                    srow[bs][row, sl] = grow[bg][row, sl] * s

        pltpu.async_copy(srow[bs], acc.at[dst_v.at[jj]], ssem[bs],
                         add=True)

        if prefetch:
            @pl.when(jj + NBUF_G < SCH)
            def _():
                pltpu.async_copy(h_hbm.at[src_v.at[jj + NBUF_G]],
                                 grow[bg], gsem[bg])

    @pl.loop(0, NSTG)
    def _(stage):
        pltpu.sync_copy(src_hbm.at[cid, sid, pl.ds(stage * SCH, SCH)], src_v)
        pltpu.sync_copy(dst_hbm.at[cid, sid, pl.ds(stage * SCH, SCH)], dst_v)
        pltpu.sync_copy(
            vals_hbm.at[cid, sid, pl.ds(stage * SCH * CHUNK, SCH * CHUNK)],
            vals_v)

        for b in range(NBUF_G):
            pltpu.async_copy(h_hbm.at[src_v.at[b]], grow[b], gsem[b])

        @pl.loop(0, NMAIN, step=NBUF_G)
        def _(j):
            for b in range(NBUF_G):
                chunk_body(j + b, b, b % NBUF_S, True)

        for t in range(NTAIL):
            jj = NMAIN + t
            chunk_body(jj, jj % NBUF_G, jj % NBUF_S, False)

        for b in range(NBUF_S):
            jj = SCH - NBUF_S + b
            pltpu.make_async_copy(srow[jj % NBUF_S], acc.at[dst_v.at[jj]],
                                  ssem[jj % NBUF_S]).wait()

    plsc.subcore_barrier()
    pltpu.sync_copy(acc.at[rows], out_hbm.at[cid, rows])

    @pl.when(sid == 0)
    def _():
        pltpu.sync_copy(acc.at[rem], out_hbm.at[cid, rem])


def _mm_body(x_ref, w_ref, o_ref):
    o_ref[...] = jnp.dot(x_ref[...], w_ref[...],
                         preferred_element_type=jnp.float32)


def _mm_combine_body(p_ref, w_ref, o_ref):
    x = jnp.maximum(p_ref[0] + p_ref[1], 0.0)
    o_ref[...] = jnp.dot(x, w_ref[...], preferred_element_type=jnp.float32)


def _combine_body(q_ref, o_ref):
    o_ref[...] = jnp.maximum(q_ref[0] + q_ref[1], 0.0)


_full_shape = jax.ShapeDtypeStruct((N, D), jnp.float32)
_mm = pl.pallas_call(_mm_body, out_shape=_full_shape)
_mm_combine = pl.pallas_call(_mm_combine_body, out_shape=_full_shape)
_combine = pl.pallas_call(_combine_body, out_shape=_full_shape)


def kernel(node_feats, edge_index, edge_vals, nodes_mask, W0, W1):
    src = edge_index[0].reshape(NC, NS, NCHUNK, CHUNK)
    dst = edge_index[1].reshape(NC, NS, NCHUNK, CHUNK)
    vals = edge_vals.reshape(NC, NS, EPT)
    zeros = jnp.zeros((N, D), jnp.float32)

    h0 = _mm(node_feats, W0)
    p = _spmm_sc(h0, src, dst, vals, zeros)
    h1 = _mm_combine(p, W1)
    q = _spmm_sc(h1, src, dst, vals, zeros)
    return _combine(q)
